# baseline (speedup 1.0000x reference)
import functools

import jax
import jax.numpy as jnp
from jax import lax
from jax.experimental import pallas as pl
from jax.experimental.pallas import tpu as pltpu
from jax.experimental.pallas import tpu_sc as plsc

_NUM_CORES = 2
_NUM_SUBCORES = 16
_LANES = 16
_CHUNK = 128


def _embedding_bag_sc(indices_t, table, batch, hist, dim):
    n_workers = _NUM_CORES * _NUM_SUBCORES
    bags_per_w = batch // n_workers
    bags_per_core = bags_per_w * _NUM_SUBCORES

    mesh = plsc.VectorSubcoreMesh(core_axis_name="c", subcore_axis_name="s")

    @functools.partial(
        pl.kernel,
        out_type=jax.ShapeDtypeStruct((batch, dim), jnp.float32),
        mesh=mesh,
        scratch_types=[
            pltpu.VMEM((hist, _CHUNK), jnp.int32),
            pltpu.VMEM((_CHUNK,), jnp.int32),
            pltpu.VMEM((2, _CHUNK, dim), jnp.float32),
            pltpu.VMEM_SHARED((bags_per_core, dim), jnp.float32),
            pltpu.SemaphoreType.DMA,
            pltpu.SemaphoreType.DMA,
        ],
        compiler_params=pltpu.CompilerParams(use_tc_tiling_on_sc=False),
    )
    def ker(idx_hbm, table_hbm, out_hbm, idx_v, bag_v, rows_v, acc_sh,
            sem0, sem1):
        c = lax.axis_index("c")
        s = lax.axis_index("s")
        w = c * _NUM_SUBCORES + s
        b0 = w * bags_per_w

        pltpu.sync_copy(idx_hbm.at[:, pl.ds(b0, bags_per_w)], idx_v)

        sbase = s * bags_per_w
        for j in range(0, _CHUNK, _LANES):
            bag_v[pl.ds(j, _LANES)] = lax.iota(jnp.int32, _LANES) + (
                sbase + j)

        @pl.loop(0, _CHUNK)
        def _(b):
            for d in range(0, dim, _LANES):
                rows_v[0, b, pl.ds(d, _LANES)] = jnp.zeros((_LANES,),
                                                           jnp.float32)
        for z in range(0, bags_per_w, _CHUNK):
            pltpu.sync_copy(rows_v.at[0],
                            acc_sh.at[pl.ds(sbase + z, _CHUNK)])

        sems = (sem0, sem1)
        descs = [None, None]
        for l in range(hist):
            b = l % 2
            descs[b] = pltpu.async_copy(
                table_hbm.at[idx_v.at[l]], rows_v.at[b], sems[b])
            if l >= 1:
                pb = (l - 1) % 2
                descs[pb].wait()
                pltpu.sync_copy(rows_v.at[pb], acc_sh.at[bag_v], add=True)
        lb = (hist - 1) % 2
        descs[lb].wait()
        pltpu.sync_copy(rows_v.at[lb], acc_sh.at[bag_v], add=True)

        pltpu.sync_copy(acc_sh.at[pl.ds(sbase, bags_per_w)],
                        out_hbm.at[pl.ds(b0, bags_per_w)])

    return ker(indices_t, table)


def _relayout_tc(table_t, vocab, dim):
    cols = 8192
    packs = 128 // dim
    rows = cols // packs
    grid = (vocab + cols - 1) // cols

    def body(x_ref, o_ref):
        x3 = x_ref[...].reshape(dim, rows, packs)
        t3 = jnp.transpose(x3, (1, 2, 0))
        o_ref[...] = jnp.concatenate(
            [t3[:, k, :] for k in range(packs)], axis=1)

    return pl.pallas_call(
        body,
        grid=(grid,),
        in_specs=[pl.BlockSpec((dim, cols), lambda i: (0, i))],
        out_specs=pl.BlockSpec((rows, 128), lambda i: (i, 0)),
        out_shape=jax.ShapeDtypeStruct((vocab * dim // 128, 128),
                                       jnp.float32),
        compiler_params=pltpu.CompilerParams(
            dimension_semantics=("parallel",)),
    )(table_t)


def kernel(indices, table):
    batch, hist = indices.shape
    vocab, dim = table.shape
    n_workers = _NUM_CORES * _NUM_SUBCORES
    assert batch % (n_workers * _CHUNK) == 0 and dim % _LANES == 0
    assert batch // n_workers == _CHUNK
    assert 128 % dim == 0 and (vocab * dim) % 128 == 0

    indices_t = indices.T.astype(jnp.int32)
    table_lin = _relayout_tc(table.astype(jnp.float32).T, vocab, dim)
    return _embedding_bag_sc(indices_t, table_lin.reshape(vocab, dim),
                             batch, hist, dim)


# speedup vs baseline: 8.2543x; 8.2543x over previous
import functools

import jax
import jax.numpy as jnp
from jax import lax
from jax.experimental import pallas as pl
from jax.experimental.pallas import tpu as pltpu
from jax.experimental.pallas import tpu_sc as plsc

_NUM_CORES = 2
_NUM_SUBCORES = 16
_LANES = 16
_CHUNK = 128


def _embedding_bag_sc(indices_t, table_pad, batch, hist, dim):
    n_workers = _NUM_CORES * _NUM_SUBCORES
    bags_per_w = batch // n_workers
    bags_per_core = bags_per_w * _NUM_SUBCORES

    mesh = plsc.VectorSubcoreMesh(core_axis_name="c", subcore_axis_name="s")

    @functools.partial(
        pl.kernel,
        out_type=jax.ShapeDtypeStruct((batch, dim), jnp.float32),
        mesh=mesh,
        scratch_types=[
            pltpu.VMEM((hist, _CHUNK), jnp.int32),
            pltpu.VMEM((_CHUNK,), jnp.int32),
            pltpu.VMEM((2, _CHUNK, 128), jnp.float32),
            pltpu.VMEM_SHARED((bags_per_core, 128), jnp.float32),
            pltpu.SemaphoreType.DMA,
            pltpu.SemaphoreType.DMA,
        ],
        compiler_params=pltpu.CompilerParams(use_tc_tiling_on_sc=False),
    )
    def ker(idx_hbm, table_hbm, out_hbm, idx_v, bag_v, rows_v, acc_sh,
            sem0, sem1):
        c = lax.axis_index("c")
        s = lax.axis_index("s")
        w = c * _NUM_SUBCORES + s
        b0 = w * bags_per_w

        pltpu.sync_copy(idx_hbm.at[:, pl.ds(b0, bags_per_w)], idx_v)

        sbase = s * bags_per_w
        for j in range(0, _CHUNK, _LANES):
            bag_v[pl.ds(j, _LANES)] = lax.iota(jnp.int32, _LANES) + (
                sbase + j)

        @pl.loop(0, _CHUNK)
        def _(b):
            for d in range(0, 128, _LANES):
                rows_v[0, b, pl.ds(d, _LANES)] = jnp.zeros((_LANES,),
                                                           jnp.float32)
        for z in range(0, bags_per_w, _CHUNK):
            pltpu.sync_copy(rows_v.at[0],
                            acc_sh.at[pl.ds(sbase + z, _CHUNK)])

        sems = (sem0, sem1)
        descs = [None, None]
        for l in range(hist):
            b = l % 2
            descs[b] = pltpu.async_copy(
                table_hbm.at[idx_v.at[l]], rows_v.at[b], sems[b])
            if l >= 1:
                pb = (l - 1) % 2
                descs[pb].wait()
                pltpu.sync_copy(rows_v.at[pb], acc_sh.at[bag_v], add=True)
        lb = (hist - 1) % 2
        descs[lb].wait()
        pltpu.sync_copy(rows_v.at[lb], acc_sh.at[bag_v], add=True)

        pltpu.sync_copy(acc_sh.at[pl.ds(sbase, bags_per_w), pl.ds(0, dim)],
                        out_hbm.at[pl.ds(b0, bags_per_w)])

    return ker(indices_t, table_pad)


def _relayout_tc(table_t, vocab, dim):
    cols = 8192
    grid = (vocab + cols - 1) // cols

    def body(x_ref, o_ref):
        t = x_ref[...].T
        o_ref[...] = jnp.pad(t, ((0, 0), (0, 128 - dim)))

    return pl.pallas_call(
        body,
        grid=(grid,),
        in_specs=[pl.BlockSpec((dim, cols), lambda i: (0, i))],
        out_specs=pl.BlockSpec((cols, 128), lambda i: (i, 0)),
        out_shape=jax.ShapeDtypeStruct((vocab, 128), jnp.float32),
        compiler_params=pltpu.CompilerParams(
            dimension_semantics=("parallel",)),
    )(table_t)


def kernel(indices, table):
    batch, hist = indices.shape
    vocab, dim = table.shape
    n_workers = _NUM_CORES * _NUM_SUBCORES
    assert batch % (n_workers * _CHUNK) == 0 and dim % _LANES == 0
    assert batch // n_workers == _CHUNK
    assert dim <= 128

    indices_t = indices.T.astype(jnp.int32)
    table_pad = _relayout_tc(table.astype(jnp.float32).T, vocab, dim)
    return _embedding_bag_sc(indices_t, table_pad, batch, hist, dim)


# speedup vs baseline: 10.2287x; 1.2392x over previous
import functools

import jax
import jax.numpy as jnp
from jax import lax
from jax.experimental import pallas as pl
from jax.experimental.pallas import tpu as pltpu
from jax.experimental.pallas import tpu_sc as plsc

_NUM_CORES = 2
_NUM_SUBCORES = 16
_LANES = 16
_CHUNK = 128


def _embedding_bag_sc(indices_t, table_lin, batch, hist, dim):
    n_workers = _NUM_CORES * _NUM_SUBCORES
    bags_per_w = batch // n_workers
    bags_per_core = bags_per_w * _NUM_SUBCORES

    mesh = plsc.VectorSubcoreMesh(core_axis_name="c", subcore_axis_name="s")

    @functools.partial(
        pl.kernel,
        out_type=jax.ShapeDtypeStruct((batch, dim), jnp.float32),
        mesh=mesh,
        scratch_types=[
            pltpu.VMEM((hist, _CHUNK), jnp.int32),
            pltpu.VMEM((_CHUNK,), jnp.int32),
            pltpu.VMEM((2, _CHUNK, dim), jnp.float32),
            pltpu.VMEM_SHARED((bags_per_core, dim), jnp.float32),
            pltpu.SemaphoreType.DMA,
            pltpu.SemaphoreType.DMA,
        ],
        compiler_params=pltpu.CompilerParams(use_tc_tiling_on_sc=False),
    )
    def ker(idx_hbm, table_hbm, out_hbm, idx_v, bag_v, rows_v, acc_sh,
            sem0, sem1):
        c = lax.axis_index("c")
        s = lax.axis_index("s")
        w = c * _NUM_SUBCORES + s
        b0 = w * bags_per_w

        pltpu.sync_copy(idx_hbm.at[:, pl.ds(b0, bags_per_w)], idx_v)

        sbase = s * bags_per_w
        for j in range(0, _CHUNK, _LANES):
            bag_v[pl.ds(j, _LANES)] = lax.iota(jnp.int32, _LANES) + (
                sbase + j)

        @pl.loop(0, _CHUNK)
        def _(b):
            for d in range(0, dim, _LANES):
                rows_v[0, b, pl.ds(d, _LANES)] = jnp.zeros((_LANES,),
                                                           jnp.float32)
        for z in range(0, bags_per_w, _CHUNK):
            pltpu.sync_copy(rows_v.at[0],
                            acc_sh.at[pl.ds(sbase + z, _CHUNK)])

        sems = (sem0, sem1)
        descs = [None, None]
        for l in range(hist):
            b = l % 2
            descs[b] = pltpu.async_copy(
                table_hbm.at[idx_v.at[l]], rows_v.at[b], sems[b])
            if l >= 1:
                pb = (l - 1) % 2
                descs[pb].wait()
                pltpu.sync_copy(rows_v.at[pb], acc_sh.at[bag_v], add=True)
        lb = (hist - 1) % 2
        descs[lb].wait()
        pltpu.sync_copy(rows_v.at[lb], acc_sh.at[bag_v], add=True)

        pltpu.sync_copy(acc_sh.at[pl.ds(sbase, bags_per_w)],
                        out_hbm.at[pl.ds(b0, bags_per_w)])

    return ker(indices_t, table_lin)


def _relayout_tc(table_t, vocab, dim):
    cols = 16384
    n_blocks = (vocab + cols - 1) // cols
    vocab_pad = n_blocks * cols

    def body(x_ref, o_ref):
        o_ref[:, 0:dim] = x_ref[...].T

    return pl.pallas_call(
        body,
        grid=(n_blocks,),
        in_specs=[pl.BlockSpec((dim, cols), lambda i: (0, i))],
        out_specs=pl.BlockSpec((cols, 128), lambda i: (i, 0)),
        out_shape=jax.ShapeDtypeStruct((vocab_pad, 128), jnp.float32),
        compiler_params=pltpu.CompilerParams(
            dimension_semantics=("arbitrary",)),
    )(table_t)


def kernel(indices, table):
    batch, hist = indices.shape
    vocab, dim = table.shape
    n_workers = _NUM_CORES * _NUM_SUBCORES
    assert batch % (n_workers * _CHUNK) == 0 and dim % _LANES == 0
    assert batch // n_workers == _CHUNK
    assert 128 % dim == 0
    packs = 128 // dim

    indices_t = indices.T.astype(jnp.int32) * packs
    table_pad = _relayout_tc(table.astype(jnp.float32).T, vocab, dim)
    table_lin = table_pad.reshape(table_pad.shape[0] * packs, dim)
    return _embedding_bag_sc(indices_t, table_lin, batch, hist, dim)


# speedup vs baseline: 19.3115x; 1.8880x over previous
import functools

import jax
import jax.numpy as jnp
from jax import lax
from jax.experimental import pallas as pl
from jax.experimental.pallas import tpu as pltpu
from jax.experimental.pallas import tpu_sc as plsc

_NUM_CORES = 2
_NUM_SUBCORES = 16
_LANES = 16
_CHUNK = 128


def _embedding_bag_sc(indices_t, table_lin, batch, hist, dim):
    n_workers = _NUM_CORES * _NUM_SUBCORES
    bags_per_w = batch // n_workers
    bags_per_core = bags_per_w * _NUM_SUBCORES

    mesh = plsc.VectorSubcoreMesh(core_axis_name="c", subcore_axis_name="s")

    @functools.partial(
        pl.kernel,
        out_type=jax.ShapeDtypeStruct((batch, dim), jnp.float32),
        mesh=mesh,
        scratch_types=[
            pltpu.VMEM((hist, _CHUNK), jnp.int32),
            pltpu.VMEM((_CHUNK,), jnp.int32),
            pltpu.VMEM((2, _CHUNK, dim), jnp.float32),
            pltpu.VMEM_SHARED((bags_per_core, dim), jnp.float32),
            pltpu.SemaphoreType.DMA,
            pltpu.SemaphoreType.DMA,
        ],
        compiler_params=pltpu.CompilerParams(use_tc_tiling_on_sc=False),
    )
    def ker(idx_hbm, table_hbm, out_hbm, idx_v, bag_v, rows_v, acc_sh,
            sem0, sem1):
        c = lax.axis_index("c")
        s = lax.axis_index("s")
        w = c * _NUM_SUBCORES + s
        b0 = w * bags_per_w

        pltpu.sync_copy(idx_hbm.at[:, pl.ds(b0, bags_per_w)], idx_v)

        sbase = s * bags_per_w
        for j in range(0, _CHUNK, _LANES):
            bag_v[pl.ds(j, _LANES)] = lax.iota(jnp.int32, _LANES) + (
                sbase + j)

        @pl.loop(0, _CHUNK)
        def _(b):
            for d in range(0, dim, _LANES):
                rows_v[0, b, pl.ds(d, _LANES)] = jnp.zeros((_LANES,),
                                                           jnp.float32)
        for z in range(0, bags_per_w, _CHUNK):
            pltpu.sync_copy(rows_v.at[0],
                            acc_sh.at[pl.ds(sbase + z, _CHUNK)])

        sems = (sem0, sem1)
        descs = [None, None]
        for l in range(hist):
            b = l % 2
            descs[b] = pltpu.async_copy(
                table_hbm.at[idx_v.at[l]], rows_v.at[b], sems[b])
            if l >= 1:
                pb = (l - 1) % 2
                descs[pb].wait()
                pltpu.sync_copy(rows_v.at[pb], acc_sh.at[bag_v], add=True)
        lb = (hist - 1) % 2
        descs[lb].wait()
        pltpu.sync_copy(rows_v.at[lb], acc_sh.at[bag_v], add=True)

        pltpu.sync_copy(acc_sh.at[pl.ds(sbase, bags_per_w)],
                        out_hbm.at[pl.ds(b0, bags_per_w)])

    return ker(indices_t, table_lin)


def _relayout_tc(table_t, vocab, dim):
    packs = 128 // dim
    cols = 8192
    per_q = (vocab + packs * cols - 1) // (packs * cols)
    stride = per_q * cols

    def body(*refs):
        o_ref = refs[packs]
        stacked = jnp.concatenate([refs[m][...] for m in range(packs)],
                                  axis=0)
        o_ref[...] = stacked.T

    return pl.pallas_call(
        body,
        grid=(per_q,),
        in_specs=[pl.BlockSpec((dim, cols),
                               functools.partial(
                                   lambda q, i: (0, jnp.minimum(
                                       q * per_q + i,
                                       (vocab - 1) // cols)), m))
                  for m in range(packs)],
        out_specs=pl.BlockSpec((cols, 128), lambda i: (i, 0)),
        out_shape=jax.ShapeDtypeStruct((stride, 128), jnp.float32),
        compiler_params=pltpu.CompilerParams(
            dimension_semantics=("arbitrary",)),
    )(*([table_t] * packs)), stride


def kernel(indices, table):
    batch, hist = indices.shape
    vocab, dim = table.shape
    n_workers = _NUM_CORES * _NUM_SUBCORES
    assert batch % (n_workers * _CHUNK) == 0 and dim % _LANES == 0
    assert batch // n_workers == _CHUNK
    assert 128 % dim == 0
    packs = 128 // dim

    table_pack, stride = _relayout_tc(table.astype(jnp.float32).T, vocab,
                                      dim)
    it = indices.T.astype(jnp.int32)
    indices_t = (it % stride) * packs + it // stride
    table_lin = table_pack.reshape(table_pack.shape[0] * packs, dim)
    return _embedding_bag_sc(indices_t, table_lin, batch, hist, dim)


# speedup vs baseline: 19.4973x; 1.0096x over previous
import functools

import jax
import jax.numpy as jnp
from jax import lax
from jax.experimental import pallas as pl
from jax.experimental.pallas import tpu as pltpu
from jax.experimental.pallas import tpu_sc as plsc

_NUM_CORES = 2
_NUM_SUBCORES = 16
_LANES = 16
_CHUNK = 128


def _embedding_bag_sc(indices_t, table_lin, batch, hist, dim):
    n_workers = _NUM_CORES * _NUM_SUBCORES
    bags_per_w = batch // n_workers
    bags_per_core = bags_per_w * _NUM_SUBCORES

    mesh = plsc.VectorSubcoreMesh(core_axis_name="c", subcore_axis_name="s")

    @functools.partial(
        pl.kernel,
        out_type=jax.ShapeDtypeStruct((batch, dim), jnp.float32),
        mesh=mesh,
        scratch_types=[
            pltpu.VMEM((hist, _CHUNK), jnp.int32),
            pltpu.VMEM((_CHUNK,), jnp.int32),
            pltpu.VMEM((2, _CHUNK, dim), jnp.float32),
            pltpu.VMEM_SHARED((bags_per_core, dim), jnp.float32),
            pltpu.SemaphoreType.DMA,
            pltpu.SemaphoreType.DMA,
        ],
        compiler_params=pltpu.CompilerParams(use_tc_tiling_on_sc=False),
    )
    def ker(idx_hbm, table_hbm, out_hbm, idx_v, bag_v, rows_v, acc_sh,
            sem0, sem1):
        c = lax.axis_index("c")
        s = lax.axis_index("s")
        w = c * _NUM_SUBCORES + s
        b0 = w * bags_per_w

        pltpu.sync_copy(idx_hbm.at[:, pl.ds(b0, bags_per_w)], idx_v)

        sbase = s * bags_per_w
        for j in range(0, _CHUNK, _LANES):
            bag_v[pl.ds(j, _LANES)] = lax.iota(jnp.int32, _LANES) + (
                sbase + j)

        @pl.loop(0, _CHUNK)
        def _(b):
            for d in range(0, dim, _LANES):
                rows_v[0, b, pl.ds(d, _LANES)] = jnp.zeros((_LANES,),
                                                           jnp.float32)
        for z in range(0, bags_per_w, _CHUNK):
            pltpu.sync_copy(rows_v.at[0],
                            acc_sh.at[pl.ds(sbase + z, _CHUNK)])

        sems = (sem0, sem1)
        descs = [None, None]
        for l in range(hist):
            b = l % 2
            descs[b] = pltpu.async_copy(
                table_hbm.at[idx_v.at[l]], rows_v.at[b], sems[b])
            if l >= 1:
                pb = (l - 1) % 2
                descs[pb].wait()
                pltpu.sync_copy(rows_v.at[pb], acc_sh.at[bag_v], add=True)
        lb = (hist - 1) % 2
        descs[lb].wait()
        pltpu.sync_copy(rows_v.at[lb], acc_sh.at[bag_v], add=True)

        pltpu.sync_copy(acc_sh.at[pl.ds(sbase, bags_per_w)],
                        out_hbm.at[pl.ds(b0, bags_per_w)])

    return ker(indices_t, table_lin)


def _relayout_tc(table_t, vocab, dim):
    packs = 128 // dim
    cols = 16384
    per_q = (vocab + packs * cols - 1) // (packs * cols)
    stride = per_q * cols

    def body(*refs):
        o_ref = refs[packs]
        stacked = jnp.concatenate([refs[m][...] for m in range(packs)],
                                  axis=0)
        o_ref[...] = stacked.T

    return pl.pallas_call(
        body,
        grid=(per_q,),
        in_specs=[pl.BlockSpec((dim, cols),
                               functools.partial(
                                   lambda q, i: (0, jnp.minimum(
                                       q * per_q + i,
                                       (vocab - 1) // cols)), m))
                  for m in range(packs)],
        out_specs=pl.BlockSpec((cols, 128), lambda i: (i, 0)),
        out_shape=jax.ShapeDtypeStruct((stride, 128), jnp.float32),
        compiler_params=pltpu.CompilerParams(
            dimension_semantics=("arbitrary",)),
    )(*([table_t] * packs)), stride


def kernel(indices, table):
    batch, hist = indices.shape
    vocab, dim = table.shape
    n_workers = _NUM_CORES * _NUM_SUBCORES
    assert batch % (n_workers * _CHUNK) == 0 and dim % _LANES == 0
    assert batch // n_workers == _CHUNK
    assert 128 % dim == 0
    packs = 128 // dim

    table_pack, stride = _relayout_tc(table.astype(jnp.float32).T, vocab,
                                      dim)
    it = indices.T.astype(jnp.int32)
    indices_t = (it % stride) * packs + it // stride
    table_lin = table_pack.reshape(table_pack.shape[0] * packs, dim)
    return _embedding_bag_sc(indices_t, table_lin, batch, hist, dim)


# speedup vs baseline: 20.1030x; 1.0311x over previous
import functools

import jax
import jax.numpy as jnp
from jax import lax
from jax.experimental import pallas as pl
from jax.experimental.pallas import tpu as pltpu
from jax.experimental.pallas import tpu_sc as plsc

_NUM_CORES = 2
_NUM_SUBCORES = 16
_LANES = 16
_CHUNK = 128


def _embedding_bag_sc(indices_t, table_lin, batch, hist, dim):
    n_workers = _NUM_CORES * _NUM_SUBCORES
    bags_per_w = batch // n_workers
    bags_per_core = bags_per_w * _NUM_SUBCORES

    mesh = plsc.VectorSubcoreMesh(core_axis_name="c", subcore_axis_name="s")

    @functools.partial(
        pl.kernel,
        out_type=jax.ShapeDtypeStruct((batch, dim), jnp.float32),
        mesh=mesh,
        scratch_types=[
            pltpu.VMEM((hist, _CHUNK), jnp.int32),
            pltpu.VMEM((_CHUNK,), jnp.int32),
            pltpu.VMEM((4, _CHUNK, dim), jnp.float32),
            pltpu.VMEM_SHARED((bags_per_core, dim), jnp.float32),
            [pltpu.SemaphoreType.DMA] * 4,
            [pltpu.SemaphoreType.DMA] * 4,
        ],
        compiler_params=pltpu.CompilerParams(use_tc_tiling_on_sc=False),
    )
    def ker(idx_hbm, table_hbm, out_hbm, idx_v, bag_v, rows_v, acc_sh,
            gsems, ssems):
        c = lax.axis_index("c")
        s = lax.axis_index("s")
        w = c * _NUM_SUBCORES + s
        b0 = w * bags_per_w

        pltpu.sync_copy(idx_hbm.at[:, pl.ds(b0, bags_per_w)], idx_v)

        sbase = s * bags_per_w
        for j in range(0, _CHUNK, _LANES):
            bag_v[pl.ds(j, _LANES)] = lax.iota(jnp.int32, _LANES) + (
                sbase + j)

        @pl.loop(0, _CHUNK)
        def _(b):
            for d in range(0, dim, _LANES):
                rows_v[0, b, pl.ds(d, _LANES)] = jnp.zeros((_LANES,),
                                                           jnp.float32)
        for z in range(0, bags_per_w, _CHUNK):
            pltpu.sync_copy(rows_v.at[0],
                            acc_sh.at[pl.ds(sbase + z, _CHUNK)])

        gd = [None] * 4
        sd = [None] * 4
        for l in range(hist):
            b = l % 4
            if l >= 4:
                sd[b].wait()
            gd[b] = pltpu.async_copy(
                table_hbm.at[idx_v.at[l]], rows_v.at[b], gsems[b])
            if l >= 1:
                pb = (l - 1) % 4
                gd[pb].wait()
                sd[pb] = pltpu.async_copy(
                    rows_v.at[pb], acc_sh.at[bag_v], ssems[pb], add=True)
        lb = (hist - 1) % 4
        gd[lb].wait()
        sd[lb] = pltpu.async_copy(
            rows_v.at[lb], acc_sh.at[bag_v], ssems[lb], add=True)
        for b in range(4):
            if sd[b] is not None:
                sd[b].wait()

        pltpu.sync_copy(acc_sh.at[pl.ds(sbase, bags_per_w)],
                        out_hbm.at[pl.ds(b0, bags_per_w)])

    return ker(indices_t, table_lin)


def _relayout_tc(table_t, vocab, dim):
    packs = 128 // dim
    cols = 16384
    per_q = (vocab + packs * cols - 1) // (packs * cols)
    stride = per_q * cols

    def body(*refs):
        o_ref = refs[packs]
        stacked = jnp.concatenate([refs[m][...] for m in range(packs)],
                                  axis=0)
        o_ref[...] = stacked.T

    return pl.pallas_call(
        body,
        grid=(per_q,),
        in_specs=[pl.BlockSpec((dim, cols),
                               functools.partial(
                                   lambda q, i: (0, jnp.minimum(
                                       q * per_q + i,
                                       (vocab - 1) // cols)), m))
                  for m in range(packs)],
        out_specs=pl.BlockSpec((cols, 128), lambda i: (i, 0)),
        out_shape=jax.ShapeDtypeStruct((stride, 128), jnp.float32),
        compiler_params=pltpu.CompilerParams(
            dimension_semantics=("arbitrary",)),
    )(*([table_t] * packs)), stride


def kernel(indices, table):
    batch, hist = indices.shape
    vocab, dim = table.shape
    n_workers = _NUM_CORES * _NUM_SUBCORES
    assert batch % (n_workers * _CHUNK) == 0 and dim % _LANES == 0
    assert batch // n_workers == _CHUNK
    assert 128 % dim == 0
    packs = 128 // dim

    table_pack, stride = _relayout_tc(table.astype(jnp.float32).T, vocab,
                                      dim)
    it = indices.T.astype(jnp.int32)
    indices_t = (it % stride) * packs + it // stride
    table_lin = table_pack.reshape(table_pack.shape[0] * packs, dim)
    return _embedding_bag_sc(indices_t, table_lin, batch, hist, dim)


# speedup vs baseline: 20.7082x; 1.0301x over previous
import functools

import jax
import jax.numpy as jnp
from jax import lax
from jax.experimental import pallas as pl
from jax.experimental.pallas import tpu as pltpu
from jax.experimental.pallas import tpu_sc as plsc

_NUM_CORES = 2
_NUM_SUBCORES = 16
_LANES = 16
_CHUNK = 128


def _embedding_bag_sc(indices_t, table_lin, batch, hist, dim):
    n_workers = _NUM_CORES * _NUM_SUBCORES
    bags_per_w = batch // n_workers
    bags_per_core = bags_per_w * _NUM_SUBCORES

    mesh = plsc.VectorSubcoreMesh(core_axis_name="c", subcore_axis_name="s")

    @functools.partial(
        pl.kernel,
        out_type=jax.ShapeDtypeStruct((batch, dim), jnp.float32),
        mesh=mesh,
        scratch_types=[
            pltpu.VMEM((hist, _CHUNK), jnp.int32),
            pltpu.VMEM((_CHUNK,), jnp.int32),
            pltpu.VMEM((8, _CHUNK, dim), jnp.float32),
            pltpu.VMEM_SHARED((bags_per_core, dim), jnp.float32),
            [pltpu.SemaphoreType.DMA] * 8,
            [pltpu.SemaphoreType.DMA] * 8,
        ],
        compiler_params=pltpu.CompilerParams(use_tc_tiling_on_sc=False),
    )
    def ker(idx_hbm, table_hbm, out_hbm, idx_v, bag_v, rows_v, acc_sh,
            gsems, ssems):
        c = lax.axis_index("c")
        s = lax.axis_index("s")
        w = c * _NUM_SUBCORES + s
        b0 = w * bags_per_w

        pltpu.sync_copy(idx_hbm.at[:, pl.ds(b0, bags_per_w)], idx_v)

        sbase = s * bags_per_w
        for j in range(0, _CHUNK, _LANES):
            bag_v[pl.ds(j, _LANES)] = lax.iota(jnp.int32, _LANES) + (
                sbase + j)

        @pl.loop(0, _CHUNK)
        def _(b):
            for d in range(0, dim, _LANES):
                rows_v[0, b, pl.ds(d, _LANES)] = jnp.zeros((_LANES,),
                                                           jnp.float32)
        for z in range(0, bags_per_w, _CHUNK):
            pltpu.sync_copy(rows_v.at[0],
                            acc_sh.at[pl.ds(sbase + z, _CHUNK)])

        depth, ahead = 8, 3
        gd = [None] * depth
        sd = [None] * depth
        for j in range(ahead):
            gd[j] = pltpu.async_copy(
                table_hbm.at[idx_v.at[j]], rows_v.at[j], gsems[j])
        for l in range(hist):
            b = l % depth
            gd[b].wait()
            sd[b] = pltpu.async_copy(
                rows_v.at[b], acc_sh.at[bag_v], ssems[b], add=True)
            nxt = l + ahead
            if nxt < hist:
                nb = nxt % depth
                if nxt >= depth:
                    sd[nb].wait()
                gd[nb] = pltpu.async_copy(
                    table_hbm.at[idx_v.at[nxt]], rows_v.at[nb], gsems[nb])
        for b in range(depth):
            sd[b].wait()

        pltpu.sync_copy(acc_sh.at[pl.ds(sbase, bags_per_w)],
                        out_hbm.at[pl.ds(b0, bags_per_w)])

    return ker(indices_t, table_lin)


def _relayout_tc(table_t, vocab, dim):
    packs = 128 // dim
    cols = 16384
    per_q = (vocab + packs * cols - 1) // (packs * cols)
    stride = per_q * cols

    def body(*refs):
        o_ref = refs[packs]
        stacked = jnp.concatenate([refs[m][...] for m in range(packs)],
                                  axis=0)
        o_ref[...] = stacked.T

    return pl.pallas_call(
        body,
        grid=(per_q,),
        in_specs=[pl.BlockSpec((dim, cols),
                               functools.partial(
                                   lambda q, i: (0, jnp.minimum(
                                       q * per_q + i,
                                       (vocab - 1) // cols)), m))
                  for m in range(packs)],
        out_specs=pl.BlockSpec((cols, 128), lambda i: (i, 0)),
        out_shape=jax.ShapeDtypeStruct((stride, 128), jnp.float32),
        compiler_params=pltpu.CompilerParams(
            dimension_semantics=("arbitrary",)),
    )(*([table_t] * packs)), stride


def kernel(indices, table):
    batch, hist = indices.shape
    vocab, dim = table.shape
    n_workers = _NUM_CORES * _NUM_SUBCORES
    assert batch % (n_workers * _CHUNK) == 0 and dim % _LANES == 0
    assert batch // n_workers == _CHUNK
    assert 128 % dim == 0
    packs = 128 // dim

    table_pack, stride = _relayout_tc(table.astype(jnp.float32).T, vocab,
                                      dim)
    it = indices.T.astype(jnp.int32)
    indices_t = (it % stride) * packs + it // stride
    table_lin = table_pack.reshape(table_pack.shape[0] * packs, dim)
    return _embedding_bag_sc(indices_t, table_lin, batch, hist, dim)
